# Initial kernel scaffold; baseline (speedup 1.0000x reference)
import jax
import jax.numpy as jnp
from jax.experimental import pallas as pl


def kernel(features, edge_index, Wc, W1, b1, Wcls):
    raise NotImplementedError("write your pallas kernel here")


# speedup vs baseline: 5.5100x; 5.5100x over previous
import functools

import jax
import jax.numpy as jnp
from jax import lax
from jax.experimental import pallas as pl
from jax.experimental.pallas import tpu as pltpu
from jax.experimental.pallas import tpu_sc as plsc

_N = 10000
_E = 160000
_D = 256
_H = 128
_NC = 2
_NT = 16
_EB = 80
_NB = _E // (_NT * _EB)
_EBS = 100
_NBS = _E // (_NT * _EBS)
_RPT = _N // _NT
_BN = 2000

_mesh = plsc.VectorSubcoreMesh(core_axis_name="c", subcore_axis_name="s")


@functools.partial(
    pl.kernel,
    out_type=jax.ShapeDtypeStruct((_NC, _N), jnp.float32),
    mesh=_mesh,
    scratch_types=[
        pltpu.VMEM((_NB, _EB), jnp.int32),
        pltpu.VMEM((_EB,), jnp.float32),
        pltpu.VMEM_SHARED((_N,), jnp.float32),
    ],
)
def _hist_k(eidx, zeros_n, deg_out, idx_v, ones_v, sh_deg):
    c = lax.axis_index("c")
    t = lax.axis_index("s")
    pltpu.sync_copy(eidx.at[c, t], idx_v)
    for k in range(_EB // 16):
        ones_v[pl.ds(k * 16, 16)] = jnp.ones((16,), jnp.float32)

    @pl.when(t == 0)
    def _zero():
        pltpu.sync_copy(zeros_n, sh_deg)

    plsc.subcore_barrier()

    def body(j, carry):
        pltpu.sync_copy(ones_v, sh_deg.at[idx_v.at[j]], add=True)
        return carry

    lax.fori_loop(0, _NB, body, 0)
    plsc.subcore_barrier()

    @pl.when(t == 0)
    def _writeout():
        pltpu.sync_copy(sh_deg, deg_out.at[c])


def _mm_body(deg_ref, f_ref, wc_ref, out_ref):
    ns = lax.rsqrt(jnp.maximum(deg_ref[:, 0:1], 1.0))
    out_ref[0] = jnp.dot(
        f_ref[...] * ns,
        wc_ref[...],
        preferred_element_type=jnp.float32,
        precision=lax.Precision.HIGHEST,
    )


_mm = pl.pallas_call(
    _mm_body,
    grid=(_NC, _N // _BN),
    in_specs=[
        pl.BlockSpec((_BN, 2), lambda c, i: (i, 0)),
        pl.BlockSpec((_BN, _D), lambda c, i: (i, 0)),
        pl.BlockSpec((_D, _H), lambda c, i: (0, c)),
    ],
    out_specs=pl.BlockSpec((1, _BN, _H), lambda c, i: (c, i, 0)),
    out_shape=jax.ShapeDtypeStruct((_NC, _N, _H), jnp.float32),
)


@functools.partial(
    pl.kernel,
    out_type=jax.ShapeDtypeStruct((_NC, _NT, _RPT, _H), jnp.float32),
    mesh=_mesh,
    scratch_types=[
        pltpu.VMEM((_NBS, _EBS), jnp.int32),
        pltpu.VMEM((_NBS, _EBS), jnp.int32),
        pltpu.VMEM((_EBS, _H), jnp.float32),
        pltpu.VMEM_SHARED((_N, _H), jnp.float32),
        pltpu.SemaphoreType.DMA,
    ],
)
def _scatter_k(g_r, src2, dst2, zeros_nh, agg_out,
               sidx, didx, bufa, sh_acc, sema):
    c = lax.axis_index("c")
    t = lax.axis_index("s")
    pltpu.sync_copy(src2.at[c, t], sidx)
    pltpu.sync_copy(dst2.at[t], didx)
    pltpu.sync_copy(zeros_nh.at[t],
                    sh_acc.at[pl.ds(t * _RPT, _RPT)])
    plsc.subcore_barrier()

    def body(j, carry):
        pltpu.async_copy(g_r.at[sidx.at[j]], bufa, sema).wait()
        pltpu.sync_copy(bufa, sh_acc.at[didx.at[j]], add=True)
        return carry

    lax.fori_loop(0, _NBS, body, 0)

    plsc.subcore_barrier()
    pltpu.sync_copy(sh_acc.at[pl.ds(t * _RPT, _RPT)], agg_out.at[c, t])


def _fin_body(agg_ref, deg_ref, w1_ref, b1_ref, wcls_ref, out_ref, acc_ref):
    c = pl.program_id(0)
    i = pl.program_id(1)
    nd = lax.rsqrt(jnp.maximum(deg_ref[:, 1:2], 1.0))
    x = agg_ref[0] * nd
    x = jnp.where(x >= 0, x, 0.01 * x)
    s = jnp.sum(x, axis=0, keepdims=True)

    @pl.when((c == 0) & (i == 0))
    def _():
        acc_ref[0:1, 0:_H] = s

    @pl.when((c == 0) & (i > 0))
    def _():
        acc_ref[0:1, 0:_H] += s

    @pl.when((c == 1) & (i == 0))
    def _():
        acc_ref[0:1, _H:_D] = s

    @pl.when((c == 1) & (i > 0))
    def _():
        acc_ref[0:1, _H:_D] += s

    @pl.when((c == _NC - 1) & (i == _N // _BN - 1))
    def _finish():
        m = acc_ref[0:1, :]
        mm = lax.dot_general(
            m, w1_ref[...], (((1,), (1,)), ((), ())),
            preferred_element_type=jnp.float32,
            precision=lax.Precision.HIGHEST,
        )
        pooled = mm * (1.0 / _N) + b1_ref[...]
        out_ref[...] = lax.dot_general(
            pooled, wcls_ref[...], (((1,), (1,)), ((), ())),
            preferred_element_type=jnp.float32,
            precision=lax.Precision.HIGHEST,
        )


_fin = pl.pallas_call(
    _fin_body,
    grid=(_NC, _N // _BN),
    in_specs=[
        pl.BlockSpec((1, _BN, _H), lambda c, i: (c, i, 0)),
        pl.BlockSpec((_BN, 2), lambda c, i: (i, 0)),
        pl.BlockSpec((_D // 2, _D), lambda c, i: (0, 0)),
        pl.BlockSpec((1, _H), lambda c, i: (0, 0)),
        pl.BlockSpec((10, _H), lambda c, i: (0, 0)),
    ],
    out_specs=pl.BlockSpec((1, 10), lambda c, i: (0, 0)),
    out_shape=jax.ShapeDtypeStruct((1, 10), jnp.float32),
    scratch_shapes=[pltpu.VMEM((8, _D), jnp.float32)],
)


def kernel(features, edge_index, Wc, W1, b1, Wcls):
    src = edge_index[0]
    dst = edge_index[1]
    eidx = edge_index.reshape(_NC, _NT, _NB, _EB)
    zeros_n = jnp.zeros((_N,), jnp.float32)
    deg = _hist_k(eidx, zeros_n)
    degT = deg.T
    g2 = _mm(degT, features, Wc)
    g_r = g2.reshape(_NC * _N, _H)
    src2 = (src.reshape(1, _E)
            + jnp.array([[0], [_N]], jnp.int32)).reshape(_NC, _NT, _NBS, _EBS)
    dst2 = dst.reshape(_NT, _NBS, _EBS)
    zeros_nh = jnp.zeros((_NT, _RPT, _H), jnp.float32)
    agg4 = _scatter_k(g_r, src2, dst2, zeros_nh)
    agg2 = agg4.reshape(_NC, _N, _H)
    return _fin(agg2, degT, W1, b1.reshape(1, _H), Wcls)


# speedup vs baseline: 7.6588x; 1.3900x over previous
import functools

import jax
import jax.numpy as jnp
from jax import lax
from jax.experimental import pallas as pl
from jax.experimental.pallas import tpu as pltpu
from jax.experimental.pallas import tpu_sc as plsc

_N = 10000
_E = 160000
_D = 256
_H = 128
_NC = 2
_NT = 16
_EB = 80
_NB = _E // (_NT * _EB)
_EBS = 128
_PH = 2
_WB = 40
_EPAD = _NT * _PH * _WB * _EBS
_NDIS = 8
_RPT = _N // _NT
_BN = 2000

_mesh = plsc.VectorSubcoreMesh(core_axis_name="c", subcore_axis_name="s")


@functools.partial(
    pl.kernel,
    out_type=jax.ShapeDtypeStruct((_NC, _N), jnp.float32),
    mesh=_mesh,
    scratch_types=[
        pltpu.VMEM((_NB, _EB), jnp.int32),
        pltpu.VMEM((_EB,), jnp.float32),
        pltpu.VMEM_SHARED((_N,), jnp.float32),
    ],
)
def _hist_k(eidx, zeros_n, deg_out, idx_v, ones_v, sh_deg):
    c = lax.axis_index("c")
    t = lax.axis_index("s")
    pltpu.sync_copy(eidx.at[c, t], idx_v)
    for k in range(_EB // 16):
        ones_v[pl.ds(k * 16, 16)] = jnp.ones((16,), jnp.float32)

    @pl.when(t == 0)
    def _zero():
        pltpu.sync_copy(zeros_n, sh_deg)

    plsc.subcore_barrier()

    def body(j, carry):
        pltpu.sync_copy(ones_v, sh_deg.at[idx_v.at[j]], add=True)
        return carry

    lax.fori_loop(0, _NB, body, 0)
    plsc.subcore_barrier()

    @pl.when(t == 0)
    def _writeout():
        pltpu.sync_copy(sh_deg, deg_out.at[c])


def _mm_body(deg_ref, f_ref, wc_ref, out_ref):
    ns = lax.rsqrt(jnp.maximum(deg_ref[:, 0:1], 1.0))
    out_ref[0] = jnp.dot(
        f_ref[...] * ns,
        wc_ref[...],
        preferred_element_type=jnp.float32,
        precision=lax.Precision.HIGHEST,
    )


_mm = pl.pallas_call(
    _mm_body,
    grid=(_NC, _N // _BN),
    in_specs=[
        pl.BlockSpec((_BN, 2), lambda c, i: (i, 0)),
        pl.BlockSpec((_BN, _D), lambda c, i: (i, 0)),
        pl.BlockSpec((_D, _H), lambda c, i: (0, c)),
    ],
    out_specs=pl.BlockSpec((1, _BN, _H), lambda c, i: (c, i, 0)),
    out_shape=jax.ShapeDtypeStruct((_NC, _N, _H), jnp.float32),
)


@functools.partial(
    pl.kernel,
    out_type=jax.ShapeDtypeStruct((_NC, _NT, _RPT, _H), jnp.float32),
    mesh=_mesh,
    scratch_types=[
        pltpu.VMEM((_WB, _EBS), jnp.int32),
        pltpu.VMEM((_WB, _EBS), jnp.int32),
        pltpu.VMEM((_EBS, _H), jnp.float32),
        pltpu.VMEM((_EBS, _H), jnp.float32),
        pltpu.VMEM_SHARED((_N + _NDIS, _H), jnp.float32),
        pltpu.SemaphoreType.DMA,
        pltpu.SemaphoreType.DMA,
    ],
)
def _scatter_k(g_r, src5, dst5, zeros_nh, agg_out,
               sidx, didx, bufa, bufb, sh_acc, sema, semb):
    c = lax.axis_index("c")
    t = lax.axis_index("s")
    pltpu.sync_copy(zeros_nh.at[t],
                    sh_acc.at[pl.ds(t * _RPT, _RPT)])
    plsc.subcore_barrier()

    for p in range(_PH):
        pltpu.sync_copy(src5.at[c, t, p], sidx)
        pltpu.sync_copy(dst5.at[t, p], didx)
        pltpu.async_copy(g_r.at[sidx.at[0]], bufa, sema)

        def body(j2, carry):
            j = 2 * j2
            pltpu.async_copy(g_r.at[sidx.at[j + 1]], bufb, semb)
            pltpu.make_async_copy(g_r.at[sidx.at[j]], bufa, sema).wait()
            pltpu.sync_copy(bufa, sh_acc.at[didx.at[j]], add=True)
            pltpu.async_copy(g_r.at[sidx.at[j + 2]], bufa, sema)
            pltpu.make_async_copy(g_r.at[sidx.at[j + 1]], bufb, semb).wait()
            pltpu.sync_copy(bufb, sh_acc.at[didx.at[j + 1]], add=True)
            return carry

        lax.fori_loop(0, _WB // 2 - 1, body, 0)
        pltpu.async_copy(g_r.at[sidx.at[_WB - 1]], bufb, semb)
        pltpu.make_async_copy(g_r.at[sidx.at[_WB - 2]], bufa, sema).wait()
        pltpu.sync_copy(bufa, sh_acc.at[didx.at[_WB - 2]], add=True)
        pltpu.make_async_copy(g_r.at[sidx.at[_WB - 1]], bufb, semb).wait()
        pltpu.sync_copy(bufb, sh_acc.at[didx.at[_WB - 1]], add=True)

    plsc.subcore_barrier()
    pltpu.sync_copy(sh_acc.at[pl.ds(t * _RPT, _RPT)], agg_out.at[c, t])


def _fin_body(agg_ref, deg_ref, w1_ref, b1_ref, wcls_ref, out_ref, acc_ref):
    c = pl.program_id(0)
    i = pl.program_id(1)
    nd = lax.rsqrt(jnp.maximum(deg_ref[:, 1:2], 1.0))
    x = agg_ref[0] * nd
    x = jnp.where(x >= 0, x, 0.01 * x)
    s = jnp.sum(x, axis=0, keepdims=True)

    @pl.when((c == 0) & (i == 0))
    def _():
        acc_ref[0:1, 0:_H] = s

    @pl.when((c == 0) & (i > 0))
    def _():
        acc_ref[0:1, 0:_H] += s

    @pl.when((c == 1) & (i == 0))
    def _():
        acc_ref[0:1, _H:_D] = s

    @pl.when((c == 1) & (i > 0))
    def _():
        acc_ref[0:1, _H:_D] += s

    @pl.when((c == _NC - 1) & (i == _N // _BN - 1))
    def _finish():
        m = acc_ref[0:1, :]
        mm = lax.dot_general(
            m, w1_ref[...], (((1,), (1,)), ((), ())),
            preferred_element_type=jnp.float32,
            precision=lax.Precision.HIGHEST,
        )
        pooled = mm * (1.0 / _N) + b1_ref[...]
        out_ref[...] = lax.dot_general(
            pooled, wcls_ref[...], (((1,), (1,)), ((), ())),
            preferred_element_type=jnp.float32,
            precision=lax.Precision.HIGHEST,
        )


_fin = pl.pallas_call(
    _fin_body,
    grid=(_NC, _N // _BN),
    in_specs=[
        pl.BlockSpec((1, _BN, _H), lambda c, i: (c, i, 0)),
        pl.BlockSpec((_BN, 2), lambda c, i: (i, 0)),
        pl.BlockSpec((_D // 2, _D), lambda c, i: (0, 0)),
        pl.BlockSpec((1, _H), lambda c, i: (0, 0)),
        pl.BlockSpec((10, _H), lambda c, i: (0, 0)),
    ],
    out_specs=pl.BlockSpec((1, 10), lambda c, i: (0, 0)),
    out_shape=jax.ShapeDtypeStruct((1, 10), jnp.float32),
    scratch_shapes=[pltpu.VMEM((8, _D), jnp.float32)],
)


def kernel(features, edge_index, Wc, W1, b1, Wcls):
    src = edge_index[0]
    dst = edge_index[1]
    eidx = edge_index.reshape(_NC, _NT, _NB, _EB)
    zeros_n = jnp.zeros((_N,), jnp.float32)
    deg = _hist_k(eidx, zeros_n)
    degT = deg.T
    g2 = _mm(degT, features, Wc)
    g_r = g2.reshape(_NC * _N, _H)
    ar = jnp.arange(_EPAD - _E, dtype=jnp.int32)
    src_p = jnp.concatenate([src, ar % _N])
    dst_p = jnp.concatenate([dst, _N + (ar % _NDIS)])
    src5 = (src_p.reshape(1, _EPAD)
            + jnp.array([[0], [_N]], jnp.int32)).reshape(_NC, _NT, _PH, _WB, _EBS)
    dst5 = dst_p.reshape(_NT, _PH, _WB, _EBS)
    zeros_nh = jnp.zeros((_NT, _RPT, _H), jnp.float32)
    agg4 = _scatter_k(g_r, src5, dst5, zeros_nh)
    agg2 = agg4.reshape(_NC, _N, _H)
    return _fin(agg2, degT, W1, b1.reshape(1, _H), Wcls)


# speedup vs baseline: 8.1494x; 1.0640x over previous
import functools

import jax
import jax.numpy as jnp
from jax import lax
from jax.experimental import pallas as pl
from jax.experimental.pallas import tpu as pltpu
from jax.experimental.pallas import tpu_sc as plsc

_N = 10000
_E = 160000
_D = 256
_H = 128
_NC = 2
_NT = 16
_EB = 80
_NB = _E // (_NT * _EB)
_EBS = 128
_PH = 2
_WB = 40
_EPAD = _NT * _PH * _WB * _EBS
_NDIS = 8
_RPT = _N // _NT
_BN = 2000

_mesh = plsc.VectorSubcoreMesh(core_axis_name="c", subcore_axis_name="s")


@functools.partial(
    pl.kernel,
    out_type=jax.ShapeDtypeStruct((_NC, _N), jnp.float32),
    mesh=_mesh,
    scratch_types=[
        pltpu.VMEM((_NB, _EB), jnp.int32),
        pltpu.VMEM((_EB,), jnp.float32),
        pltpu.VMEM_SHARED((_N,), jnp.float32),
        pltpu.SemaphoreType.DMA,
    ],
)
def _hist_k(eidx, zeros_n, deg_out, idx_v, ones_v, sh_deg, sem):
    c = lax.axis_index("c")
    t = lax.axis_index("s")
    pltpu.sync_copy(eidx.at[c, t], idx_v)
    for k in range(_EB // 16):
        ones_v[pl.ds(k * 16, 16)] = jnp.ones((16,), jnp.float32)

    @pl.when(t == 0)
    def _zero():
        pltpu.sync_copy(zeros_n, sh_deg)

    plsc.subcore_barrier()

    def body(j, carry):
        pltpu.async_copy(ones_v, sh_deg.at[idx_v.at[j]], sem, add=True)
        return carry

    lax.fori_loop(0, _NB, body, 0)

    def drain(j, carry):
        pltpu.make_async_copy(ones_v, sh_deg.at[idx_v.at[j]], sem).wait()
        return carry

    lax.fori_loop(0, _NB, drain, 0)
    plsc.subcore_barrier()

    @pl.when(t == 0)
    def _writeout():
        pltpu.sync_copy(sh_deg, deg_out.at[c])


def _mm_body(deg_ref, f_ref, wc_ref, out_ref):
    ns = lax.rsqrt(jnp.maximum(deg_ref[:, 0:1], 1.0))
    out_ref[0] = jnp.dot(
        f_ref[...] * ns,
        wc_ref[...],
        preferred_element_type=jnp.float32,
        precision=lax.Precision.HIGHEST,
    )


_mm = pl.pallas_call(
    _mm_body,
    grid=(_NC, _N // _BN),
    in_specs=[
        pl.BlockSpec((_BN, 2), lambda c, i: (i, 0)),
        pl.BlockSpec((_BN, _D), lambda c, i: (i, 0)),
        pl.BlockSpec((_D, _H), lambda c, i: (0, c)),
    ],
    out_specs=pl.BlockSpec((1, _BN, _H), lambda c, i: (c, i, 0)),
    out_shape=jax.ShapeDtypeStruct((_NC, _N, _H), jnp.float32),
)


@functools.partial(
    pl.kernel,
    out_type=jax.ShapeDtypeStruct((_NC, _NT, _RPT, _H), jnp.float32),
    mesh=_mesh,
    scratch_types=[
        pltpu.VMEM((_WB, _EBS), jnp.int32),
        pltpu.VMEM((_WB, _EBS), jnp.int32),
        pltpu.VMEM((_EBS, _H), jnp.float32),
        pltpu.VMEM((_EBS, _H), jnp.float32),
        pltpu.VMEM_SHARED((_N + _NDIS, _H), jnp.float32),
        pltpu.SemaphoreType.DMA,
        pltpu.SemaphoreType.DMA,
    ],
)
def _scatter_k(g_r, src5, dst5, agg_out,
               sidx, didx, bufa, bufb, sh_acc, sema, semb):
    c = lax.axis_index("c")
    t = lax.axis_index("s")

    def zero_buf(r, carry):
        for k in range(_H // 16):
            bufa[r, pl.ds(k * 16, 16)] = jnp.zeros((16,), jnp.float32)
        return carry

    lax.fori_loop(0, _EBS, zero_buf, 0)
    for k in range(_RPT // _EBS):
        pltpu.sync_copy(bufa, sh_acc.at[pl.ds(t * _RPT + k * _EBS, _EBS)])
    _REM = _RPT % _EBS
    pltpu.sync_copy(bufa.at[pl.ds(0, _REM)],
                    sh_acc.at[pl.ds(t * _RPT + _RPT - _REM, _REM)])
    plsc.subcore_barrier()

    for p in range(_PH):
        pltpu.sync_copy(src5.at[c, t, p], sidx)
        pltpu.sync_copy(dst5.at[t, p], didx)
        pltpu.async_copy(g_r.at[sidx.at[0]], bufa, sema)

        def body(j2, carry):
            j = 2 * j2
            pltpu.async_copy(g_r.at[sidx.at[j + 1]], bufb, semb)
            pltpu.make_async_copy(g_r.at[sidx.at[j]], bufa, sema).wait()
            pltpu.sync_copy(bufa, sh_acc.at[didx.at[j]], add=True)
            pltpu.async_copy(g_r.at[sidx.at[j + 2]], bufa, sema)
            pltpu.make_async_copy(g_r.at[sidx.at[j + 1]], bufb, semb).wait()
            pltpu.sync_copy(bufb, sh_acc.at[didx.at[j + 1]], add=True)
            return carry

        lax.fori_loop(0, _WB // 2 - 1, body, 0)
        pltpu.async_copy(g_r.at[sidx.at[_WB - 1]], bufb, semb)
        pltpu.make_async_copy(g_r.at[sidx.at[_WB - 2]], bufa, sema).wait()
        pltpu.sync_copy(bufa, sh_acc.at[didx.at[_WB - 2]], add=True)
        pltpu.make_async_copy(g_r.at[sidx.at[_WB - 1]], bufb, semb).wait()
        pltpu.sync_copy(bufb, sh_acc.at[didx.at[_WB - 1]], add=True)

    plsc.subcore_barrier()
    pltpu.sync_copy(sh_acc.at[pl.ds(t * _RPT, _RPT)], agg_out.at[c, t])


def _fin_body(agg_ref, deg_ref, w1_ref, b1_ref, wcls_ref, out_ref, acc_ref):
    c = pl.program_id(0)
    i = pl.program_id(1)
    nd = lax.rsqrt(jnp.maximum(deg_ref[:, 1:2], 1.0))
    x = agg_ref[0] * nd
    x = jnp.where(x >= 0, x, 0.01 * x)
    s = jnp.sum(x, axis=0, keepdims=True)

    @pl.when((c == 0) & (i == 0))
    def _():
        acc_ref[0:1, 0:_H] = s

    @pl.when((c == 0) & (i > 0))
    def _():
        acc_ref[0:1, 0:_H] += s

    @pl.when((c == 1) & (i == 0))
    def _():
        acc_ref[0:1, _H:_D] = s

    @pl.when((c == 1) & (i > 0))
    def _():
        acc_ref[0:1, _H:_D] += s

    @pl.when((c == _NC - 1) & (i == _N // _BN - 1))
    def _finish():
        m = acc_ref[0:1, :]
        mm = lax.dot_general(
            m, w1_ref[...], (((1,), (1,)), ((), ())),
            preferred_element_type=jnp.float32,
            precision=lax.Precision.HIGHEST,
        )
        pooled = mm * (1.0 / _N) + b1_ref[...]
        out_ref[...] = lax.dot_general(
            pooled, wcls_ref[...], (((1,), (1,)), ((), ())),
            preferred_element_type=jnp.float32,
            precision=lax.Precision.HIGHEST,
        )


_fin = pl.pallas_call(
    _fin_body,
    grid=(_NC, _N // _BN),
    in_specs=[
        pl.BlockSpec((1, _BN, _H), lambda c, i: (c, i, 0)),
        pl.BlockSpec((_BN, 2), lambda c, i: (i, 0)),
        pl.BlockSpec((_D // 2, _D), lambda c, i: (0, 0)),
        pl.BlockSpec((1, _H), lambda c, i: (0, 0)),
        pl.BlockSpec((10, _H), lambda c, i: (0, 0)),
    ],
    out_specs=pl.BlockSpec((1, 10), lambda c, i: (0, 0)),
    out_shape=jax.ShapeDtypeStruct((1, 10), jnp.float32),
    scratch_shapes=[pltpu.VMEM((8, _D), jnp.float32)],
)


def kernel(features, edge_index, Wc, W1, b1, Wcls):
    src = edge_index[0]
    dst = edge_index[1]
    eidx = edge_index.reshape(_NC, _NT, _NB, _EB)
    zeros_n = jnp.zeros((_N,), jnp.float32)
    deg = _hist_k(eidx, zeros_n)
    degT = deg.T
    g2 = _mm(degT, features, Wc)
    g_r = g2.reshape(_NC * _N, _H)
    ar = jnp.arange(_EPAD - _E, dtype=jnp.int32)
    src_p = jnp.concatenate([src, ar % _N])
    dst_p = jnp.concatenate([dst, _N + (ar % _NDIS)])
    src5 = (src_p.reshape(1, _EPAD)
            + jnp.array([[0], [_N]], jnp.int32)).reshape(_NC, _NT, _PH, _WB, _EBS)
    dst5 = dst_p.reshape(_NT, _PH, _WB, _EBS)
    agg4 = _scatter_k(g_r, src5, dst5)
    agg2 = agg4.reshape(_NC, _N, _H)
    return _fin(agg2, degT, W1, b1.reshape(1, _H), Wcls)


# speedup vs baseline: 8.6476x; 1.0611x over previous
import functools

import jax
import jax.numpy as jnp
from jax import lax
from jax.experimental import pallas as pl
from jax.experimental.pallas import tpu as pltpu
from jax.experimental.pallas import tpu_sc as plsc

_N = 10000
_E = 160000
_D = 256
_H = 128
_NC = 2
_NT = 16
_EB = 80
_NB = _E // (_NT * _EB)
_EBS = 128
_PH = 2
_WB = 40
_EPAD = _NT * _PH * _WB * _EBS
_NDIS = 8
_WRT = 624
_WLAST = _N - (_NT - 1) * _WRT
_RPT = _N // _NT
_BN = 2000

_mesh = plsc.VectorSubcoreMesh(core_axis_name="c", subcore_axis_name="s")


@functools.partial(
    pl.kernel,
    out_type=jax.ShapeDtypeStruct((_NC, _N), jnp.float32),
    mesh=_mesh,
    scratch_types=[
        pltpu.VMEM((_PH, _WB, _EBS), jnp.int32),
        pltpu.VMEM((_EBS,), jnp.float32),
        pltpu.VMEM_SHARED((_N,), jnp.float32),
        pltpu.SemaphoreType.DMA,
    ],
)
def _hist_k(src5, dst5, zeros_n, deg_out, idx_v, ones_v, sh_deg, sem):
    c = lax.axis_index("c")
    t = lax.axis_index("s")

    @pl.when(c == 0)
    def _load_src():
        pltpu.sync_copy(src5.at[0, t], idx_v)

    @pl.when(c == 1)
    def _load_dst():
        pltpu.sync_copy(dst5.at[t], idx_v)

    for k in range(_EBS // 16):
        ones_v[pl.ds(k * 16, 16)] = jnp.ones((16,), jnp.float32)

    @pl.when(t == 0)
    def _zero():
        pltpu.sync_copy(zeros_n, sh_deg)

    plsc.subcore_barrier()

    nb1 = jnp.where(t == _NT - 1, _WB - (_EPAD - _E) // _EBS, _WB)
    for p in range(_PH):
        hi = _WB if p == 0 else nb1

        def body(j, carry):
            pltpu.async_copy(ones_v, sh_deg.at[idx_v.at[p, j]], sem, add=True)
            return carry

        lax.fori_loop(0, hi, body, 0)
    for p in range(_PH):
        hi = _WB if p == 0 else nb1

        def drain(j, carry):
            pltpu.make_async_copy(ones_v, sh_deg.at[idx_v.at[p, j]], sem).wait()
            return carry

        lax.fori_loop(0, hi, drain, 0)
    plsc.subcore_barrier()

    @pl.when(t == 0)
    def _writeout():
        pltpu.sync_copy(sh_deg, deg_out.at[c])


def _mm_body(deg_ref, f_ref, wc_ref, out_ref):
    ns = lax.rsqrt(jnp.maximum(deg_ref[:, 0:1], 1.0))
    out_ref[0] = jnp.dot(
        f_ref[...] * ns,
        wc_ref[...],
        preferred_element_type=jnp.float32,
    )


_mm = pl.pallas_call(
    _mm_body,
    grid=(_NC, _N // _BN),
    in_specs=[
        pl.BlockSpec((_BN, 2), lambda c, i: (i, 0)),
        pl.BlockSpec((_BN, _D), lambda c, i: (i, 0)),
        pl.BlockSpec((_D, _H), lambda c, i: (0, c)),
    ],
    out_specs=pl.BlockSpec((1, _BN, _H), lambda c, i: (c, i, 0)),
    out_shape=jax.ShapeDtypeStruct((_NC, _N, _H), jnp.float32),
)


@functools.partial(
    pl.kernel,
    out_type=jax.ShapeDtypeStruct((_NC, _N, _H), jnp.float32),
    mesh=_mesh,
    scratch_types=[
        pltpu.VMEM((_WB, _EBS), jnp.int32),
        pltpu.VMEM((_WB, _EBS), jnp.int32),
        pltpu.VMEM((_EBS, _H), jnp.float32),
        pltpu.VMEM((_EBS, _H), jnp.float32),
        pltpu.VMEM_SHARED((_N + _NDIS, _H), jnp.float32),
        pltpu.SemaphoreType.DMA,
        pltpu.SemaphoreType.DMA,
    ],
)
def _scatter_k(g_r, src5, dst5, agg_out,
               sidx, didx, bufa, bufb, sh_acc, sema, semb):
    c = lax.axis_index("c")
    t = lax.axis_index("s")

    def zero_buf(r, carry):
        for k in range(_H // 16):
            bufa[r, pl.ds(k * 16, 16)] = jnp.zeros((16,), jnp.float32)
        return carry

    lax.fori_loop(0, _EBS, zero_buf, 0)
    for k in range(5):
        pltpu.sync_copy(bufa, sh_acc.at[pl.ds(t * _WRT + k * _EBS, _EBS)])
    plsc.subcore_barrier()

    for p in range(_PH):
        pltpu.sync_copy(src5.at[c, t, p], sidx)
        pltpu.sync_copy(dst5.at[t, p], didx)
        pltpu.async_copy(g_r.at[sidx.at[0]], bufa, sema)

        def body(j2, carry):
            j = 2 * j2
            pltpu.async_copy(g_r.at[sidx.at[j + 1]], bufb, semb)
            pltpu.make_async_copy(g_r.at[sidx.at[j]], bufa, sema).wait()
            pltpu.sync_copy(bufa, sh_acc.at[didx.at[j]], add=True)
            pltpu.async_copy(g_r.at[sidx.at[j + 2]], bufa, sema)
            pltpu.make_async_copy(g_r.at[sidx.at[j + 1]], bufb, semb).wait()
            pltpu.sync_copy(bufb, sh_acc.at[didx.at[j + 1]], add=True)
            return carry

        lax.fori_loop(0, _WB // 2 - 1, body, 0)
        pltpu.async_copy(g_r.at[sidx.at[_WB - 1]], bufb, semb)
        pltpu.make_async_copy(g_r.at[sidx.at[_WB - 2]], bufa, sema).wait()
        pltpu.sync_copy(bufa, sh_acc.at[didx.at[_WB - 2]], add=True)
        pltpu.make_async_copy(g_r.at[sidx.at[_WB - 1]], bufb, semb).wait()
        pltpu.sync_copy(bufb, sh_acc.at[didx.at[_WB - 1]], add=True)

    plsc.subcore_barrier()

    @pl.when(t < _NT - 1)
    def _wr():
        pltpu.sync_copy(sh_acc.at[pl.ds(t * _WRT, _WRT)],
                        agg_out.at[c, pl.ds(t * _WRT, _WRT)])

    @pl.when(t == _NT - 1)
    def _wr_last():
        pltpu.sync_copy(sh_acc.at[pl.ds((_NT - 1) * _WRT, _WLAST)],
                        agg_out.at[c, pl.ds((_NT - 1) * _WRT, _WLAST)])


def _fin_body(agg_ref, deg_ref, w1_ref, b1_ref, wcls_ref, out_ref, acc_ref):
    c = pl.program_id(0)
    i = pl.program_id(1)
    nd = lax.rsqrt(jnp.maximum(deg_ref[:, 1:2], 1.0))
    x = agg_ref[0] * nd
    x = jnp.where(x >= 0, x, 0.01 * x)
    s = lax.dot_general(
        jnp.ones((1, _BN), jnp.float32), x, (((1,), (0,)), ((), ())),
        preferred_element_type=jnp.float32,
        precision=lax.Precision.HIGHEST,
    )

    @pl.when((c == 0) & (i == 0))
    def _():
        acc_ref[0:1, 0:_H] = s

    @pl.when((c == 0) & (i > 0))
    def _():
        acc_ref[0:1, 0:_H] += s

    @pl.when((c == 1) & (i == 0))
    def _():
        acc_ref[0:1, _H:_D] = s

    @pl.when((c == 1) & (i > 0))
    def _():
        acc_ref[0:1, _H:_D] += s

    @pl.when((c == _NC - 1) & (i == _N // _BN - 1))
    def _finish():
        m = acc_ref[0:1, :]
        mm = lax.dot_general(
            m, w1_ref[...], (((1,), (1,)), ((), ())),
            preferred_element_type=jnp.float32,
            precision=lax.Precision.HIGHEST,
        )
        pooled = mm * (1.0 / _N) + b1_ref[...]
        out_ref[...] = lax.dot_general(
            pooled, wcls_ref[...], (((1,), (1,)), ((), ())),
            preferred_element_type=jnp.float32,
            precision=lax.Precision.HIGHEST,
        )


_fin = pl.pallas_call(
    _fin_body,
    grid=(_NC, _N // _BN),
    in_specs=[
        pl.BlockSpec((1, _BN, _H), lambda c, i: (c, i, 0)),
        pl.BlockSpec((_BN, 2), lambda c, i: (i, 0)),
        pl.BlockSpec((_D // 2, _D), lambda c, i: (0, 0)),
        pl.BlockSpec((1, _H), lambda c, i: (0, 0)),
        pl.BlockSpec((10, _H), lambda c, i: (0, 0)),
    ],
    out_specs=pl.BlockSpec((1, 10), lambda c, i: (0, 0)),
    out_shape=jax.ShapeDtypeStruct((1, 10), jnp.float32),
    scratch_shapes=[pltpu.VMEM((8, _D), jnp.float32)],
)


def kernel(features, edge_index, Wc, W1, b1, Wcls):
    src = edge_index[0]
    dst = edge_index[1]
    ar = jnp.arange(_EPAD - _E, dtype=jnp.int32)
    src_p = jnp.concatenate([src, ar % _N])
    dst_p = jnp.concatenate([dst, _N + (ar % _NDIS)])
    src5 = (src_p.reshape(1, _EPAD)
            + jnp.array([[0], [_N]], jnp.int32)).reshape(_NC, _NT, _PH, _WB, _EBS)
    dst5 = dst_p.reshape(_NT, _PH, _WB, _EBS)
    zeros_n = jnp.zeros((_N,), jnp.float32)
    deg = _hist_k(src5, dst5, zeros_n)
    degT = deg.T
    g2 = _mm(degT, features, Wc)
    g_r = g2.reshape(_NC * _N, _H)
    agg2 = _scatter_k(g_r, src5, dst5)
    return _fin(agg2, degT, W1, b1.reshape(1, _H), Wcls)


# speedup vs baseline: 9.2288x; 1.0672x over previous
import functools

import jax
import jax.numpy as jnp
from jax import lax
from jax.experimental import pallas as pl
from jax.experimental.pallas import tpu as pltpu
from jax.experimental.pallas import tpu_sc as plsc

_N = 10000
_E = 160000
_D = 256
_H = 128
_NC = 2
_NT = 16
_EB = 80
_NB = _E // (_NT * _EB)
_EBS = 128
_PH = 2
_WB = 40
_EPAD = _NT * _PH * _WB * _EBS
_NDIS = 8
_WRT = 624
_WLAST = _N - (_NT - 1) * _WRT
_RPT = _N // _NT
_BN = 2000

_mesh = plsc.VectorSubcoreMesh(core_axis_name="c", subcore_axis_name="s")


@functools.partial(
    pl.kernel,
    out_type=jax.ShapeDtypeStruct((_NC, _N), jnp.float32),
    mesh=_mesh,
    scratch_types=[
        pltpu.VMEM((_PH, _WB, _EBS), jnp.int32),
        pltpu.VMEM((_EBS,), jnp.float32),
        pltpu.VMEM_SHARED((_N,), jnp.float32),
        pltpu.SemaphoreType.DMA,
    ],
)
def _hist_k(src4, dst4, zeros_n, deg_out, idx_v, ones_v, sh_deg, sem):
    c = lax.axis_index("c")
    t = lax.axis_index("s")

    @pl.when(c == 0)
    def _load_src():
        pltpu.sync_copy(src4.at[t], idx_v)

    @pl.when(c == 1)
    def _load_dst():
        pltpu.sync_copy(dst4.at[t], idx_v)

    for k in range(_EBS // 16):
        ones_v[pl.ds(k * 16, 16)] = jnp.ones((16,), jnp.float32)

    @pl.when(t == 0)
    def _zero():
        pltpu.sync_copy(zeros_n, sh_deg)

    plsc.subcore_barrier()

    nb1 = jnp.where(t == _NT - 1, _WB - (_EPAD - _E) // _EBS, _WB)
    for p in range(_PH):
        hi = _WB if p == 0 else nb1

        def body(j, carry):
            pltpu.async_copy(ones_v, sh_deg.at[idx_v.at[p, j]], sem, add=True)
            return carry

        lax.fori_loop(0, hi, body, 0)
    for p in range(_PH):
        hi = _WB if p == 0 else nb1

        def drain(j, carry):
            pltpu.make_async_copy(ones_v, sh_deg.at[idx_v.at[p, j]], sem).wait()
            return carry

        lax.fori_loop(0, hi, drain, 0)
    plsc.subcore_barrier()

    @pl.when(t == 0)
    def _writeout():
        pltpu.sync_copy(sh_deg, deg_out.at[c])


def _mm_body(deg_ref, f_ref, wc_ref, out_ref):
    ns = lax.rsqrt(jnp.maximum(deg_ref[:, 0:1], 1.0))
    res = jnp.dot(
        f_ref[...] * ns,
        wc_ref[...],
        preferred_element_type=jnp.float32,
    )
    out_ref[0] = res[:, :_H]
    out_ref[1] = res[:, _H:]


_mm = pl.pallas_call(
    _mm_body,
    grid=(_N // _BN,),
    in_specs=[
        pl.BlockSpec((_BN, 2), lambda i: (i, 0)),
        pl.BlockSpec((_BN, _D), lambda i: (i, 0)),
        pl.BlockSpec((_D, _D), lambda i: (0, 0)),
    ],
    out_specs=pl.BlockSpec((_NC, _BN, _H), lambda i: (0, i, 0)),
    out_shape=jax.ShapeDtypeStruct((_NC, _N, _H), jnp.float32),
)


@functools.partial(
    pl.kernel,
    out_type=jax.ShapeDtypeStruct((_NC, _N, _H), jnp.float32),
    mesh=_mesh,
    scratch_types=[
        pltpu.VMEM((_WB, _EBS), jnp.int32),
        pltpu.VMEM((_WB, _EBS), jnp.int32),
        pltpu.VMEM((_EBS, _H), jnp.float32),
        pltpu.VMEM((_EBS, _H), jnp.float32),
        pltpu.VMEM_SHARED((_N + _NDIS, _H), jnp.float32),
        pltpu.SemaphoreType.DMA,
        pltpu.SemaphoreType.DMA,
    ],
)
def _scatter_k(g_r, src4, dst4, agg_out,
               sidx, didx, bufa, bufb, sh_acc, sema, semb):
    c = lax.axis_index("c")
    t = lax.axis_index("s")

    def zero_buf(r, carry):
        for k in range(_H // 16):
            bufa[r, pl.ds(k * 16, 16)] = jnp.zeros((16,), jnp.float32)
        return carry

    lax.fori_loop(0, _EBS, zero_buf, 0)
    for k in range(5):
        pltpu.sync_copy(bufa, sh_acc.at[pl.ds(t * _WRT + k * _EBS, _EBS)])
    plsc.subcore_barrier()

    off = c * _N
    for p in range(_PH):
        pltpu.sync_copy(src4.at[t, p], sidx)
        pltpu.sync_copy(dst4.at[t, p], didx)

        def add_off(j, carry):
            for k in range(_EBS // 16):
                sl = pl.ds(k * 16, 16)
                sidx[j, sl] = sidx[j, sl] + off
            return carry

        lax.fori_loop(0, _WB, add_off, 0)
        pltpu.async_copy(g_r.at[sidx.at[0]], bufa, sema)

        def body(j2, carry):
            j = 2 * j2
            pltpu.async_copy(g_r.at[sidx.at[j + 1]], bufb, semb)
            pltpu.make_async_copy(g_r.at[sidx.at[j]], bufa, sema).wait()
            pltpu.sync_copy(bufa, sh_acc.at[didx.at[j]], add=True)
            pltpu.async_copy(g_r.at[sidx.at[j + 2]], bufa, sema)
            pltpu.make_async_copy(g_r.at[sidx.at[j + 1]], bufb, semb).wait()
            pltpu.sync_copy(bufb, sh_acc.at[didx.at[j + 1]], add=True)
            return carry

        lax.fori_loop(0, _WB // 2 - 1, body, 0)
        pltpu.async_copy(g_r.at[sidx.at[_WB - 1]], bufb, semb)
        pltpu.make_async_copy(g_r.at[sidx.at[_WB - 2]], bufa, sema).wait()
        pltpu.sync_copy(bufa, sh_acc.at[didx.at[_WB - 2]], add=True)
        pltpu.make_async_copy(g_r.at[sidx.at[_WB - 1]], bufb, semb).wait()
        pltpu.sync_copy(bufb, sh_acc.at[didx.at[_WB - 1]], add=True)

    plsc.subcore_barrier()

    @pl.when(t < _NT - 1)
    def _wr():
        pltpu.sync_copy(sh_acc.at[pl.ds(t * _WRT, _WRT)],
                        agg_out.at[c, pl.ds(t * _WRT, _WRT)])

    @pl.when(t == _NT - 1)
    def _wr_last():
        pltpu.sync_copy(sh_acc.at[pl.ds((_NT - 1) * _WRT, _WLAST)],
                        agg_out.at[c, pl.ds((_NT - 1) * _WRT, _WLAST)])


def _fin_body(agg_ref, deg_ref, w1_ref, b1_ref, wcls_ref, out_ref, acc_ref):
    i = pl.program_id(0)
    nd = lax.rsqrt(jnp.maximum(deg_ref[:, 1:2], 1.0))
    ones = jnp.ones((1, _BN), jnp.float32)
    x0 = agg_ref[0] * nd
    x0 = jnp.where(x0 >= 0, x0, 0.01 * x0)
    x1 = agg_ref[1] * nd
    x1 = jnp.where(x1 >= 0, x1, 0.01 * x1)
    s0 = lax.dot_general(ones, x0, (((1,), (0,)), ((), ())),
                         preferred_element_type=jnp.float32)
    s1 = lax.dot_general(ones, x1, (((1,), (0,)), ((), ())),
                         preferred_element_type=jnp.float32)

    @pl.when(i == 0)
    def _():
        acc_ref[0:1, 0:_H] = s0
        acc_ref[0:1, _H:_D] = s1

    @pl.when(i > 0)
    def _():
        acc_ref[0:1, 0:_H] += s0
        acc_ref[0:1, _H:_D] += s1

    @pl.when(i == _N // _BN - 1)
    def _finish():
        m = acc_ref[0:1, :]
        mm = lax.dot_general(
            m, w1_ref[...], (((1,), (1,)), ((), ())),
            preferred_element_type=jnp.float32,
            precision=lax.Precision.HIGHEST,
        )
        pooled = mm * (1.0 / _N) + b1_ref[...]
        out_ref[...] = lax.dot_general(
            pooled, wcls_ref[...], (((1,), (1,)), ((), ())),
            preferred_element_type=jnp.float32,
            precision=lax.Precision.HIGHEST,
        )


_fin = pl.pallas_call(
    _fin_body,
    grid=(_N // _BN,),
    in_specs=[
        pl.BlockSpec((_NC, _BN, _H), lambda i: (0, i, 0)),
        pl.BlockSpec((_BN, 2), lambda i: (i, 0)),
        pl.BlockSpec((_D // 2, _D), lambda i: (0, 0)),
        pl.BlockSpec((1, _H), lambda i: (0, 0)),
        pl.BlockSpec((10, _H), lambda i: (0, 0)),
    ],
    out_specs=pl.BlockSpec((1, 10), lambda i: (0, 0)),
    out_shape=jax.ShapeDtypeStruct((1, 10), jnp.float32),
    scratch_shapes=[pltpu.VMEM((8, _D), jnp.float32)],
)


def kernel(features, edge_index, Wc, W1, b1, Wcls):
    src = edge_index[0]
    dst = edge_index[1]
    ar = jnp.arange(_EPAD - _E, dtype=jnp.int32)
    src_p = jnp.concatenate([src, ar % _N])
    dst_p = jnp.concatenate([dst, _N + (ar % _NDIS)])
    src4 = src_p.reshape(_NT, _PH, _WB, _EBS)
    dst4 = dst_p.reshape(_NT, _PH, _WB, _EBS)
    zeros_n = jnp.zeros((_N,), jnp.float32)
    deg = _hist_k(src4, dst4, zeros_n)
    degT = deg.T
    g2 = _mm(degT, features, Wc)
    g_r = g2.reshape(_NC * _N, _H)
    agg2 = _scatter_k(g_r, src4, dst4)
    return _fin(agg2, degT, W1, b1.reshape(1, _H), Wcls)
